# baseline (speedup 1.0000x reference)
import functools

import jax
import jax.numpy as jnp
from jax.experimental import pallas as pl
from jax.experimental.pallas import tpu as pltpu

_N = 10000
_E = 160000
_DIN = 128
_DK = 64
_K = 2
_ITERS = 4
_INDEX = [[0, 1], [1, 0], [1, 2], [1, 3], [1, 4], [1, 5], [1, 6], [1, 7]]

_INTERPRET = False


def _leaky(x):
    return jnp.where(x >= 0, x, 0.2 * x)


def _l2n_last(x):
    n = jnp.sqrt(jnp.sum(x * x, axis=-1, keepdims=True))
    return x / jnp.maximum(n, 1e-12)


def _fac_body(emb_ref, w_ref, out_ref):
    x = emb_ref[0]
    for k in range(_K):
        y = jnp.dot(x, w_ref[0, k], preferred_element_type=jnp.float32)
        y = _leaky(y)
        out_ref[0, k] = _l2n_last(y)


def _fac(emb, Wtk):
    bn = 1000
    grid = (8, _N // bn)
    return pl.pallas_call(
        _fac_body,
        grid=grid,
        in_specs=[
            pl.BlockSpec((1, bn, _DIN), lambda t, nb: (t, nb, 0)),
            pl.BlockSpec((1, _K, _DIN, _DK), lambda t, nb: (t, 0, 0, 0)),
        ],
        out_specs=pl.BlockSpec((1, _K, bn, _DK), lambda t, nb: (t, 0, nb, 0)),
        out_shape=jax.ShapeDtypeStruct((8, _K, _N, _DK), jnp.float32),
        interpret=_INTERPRET,
    )(emb, Wtk)


def kernel(edge_list, emb, Wtk, at, W, q_rela):
    all_emb = _fac(emb, Wtk)
    a1 = at[:, :, :_DK]
    a2 = at[:, :, _DK:]
    r_rela = jnp.full((8, _K, _N), 1.0 / _K, dtype=jnp.float32)

    for _ in range(_ITERS):
        z_list = []
        r_list = []
        for e in range(8):
            src, dst = _INDEX[e]
            u = edge_list[e, 0]
            i = edge_list[e, 1]
            ne = all_emb[src]
            oe = all_emb[dst]
            pn = jnp.einsum('knd,kd->kn', ne, a1[e])
            po = jnp.einsum('knd,kd->kn', oe, a2[e])
            ets = pn[:, u] + po[:, i]
            ev = jnp.sum(jnp.maximum(ets, 0.0) * r_rela[e][:, u], axis=0)
            w = jnp.exp(ev)
            s = jax.ops.segment_sum(w, u, num_segments=_N)
            num = jnp.stack([
                jax.ops.segment_sum(w[:, None] * oe[k, i, :], u,
                                    num_segments=_N)
                for k in range(_K)
            ])
            z = num / jnp.maximum(s, 1e-30)[None, :, None]
            emb_z = jnp.einsum('knd,df->knf', _leaky(z), W)
            nr = jnp.einsum('knd,d->kn', jnp.tanh(emb_z), q_rela[e])
            r = jax.nn.softmax(nr, axis=0)
            z_list.append(emb_z)
            r_list.append(r)
        ego0 = all_emb[0] + z_list[0] * r_list[0][:, :, None]
        ego1 = all_emb[1]
        for j in range(1, 8):
            ego1 = ego1 + z_list[j] * r_list[j][:, :, None]
        all_emb = all_emb.at[0].set(_l2n_last(ego0))
        all_emb = all_emb.at[1].set(_l2n_last(ego1))
        r_rela = jnp.stack(r_list)

    emb_out = jnp.concatenate([all_emb[:, 0], all_emb[:, 1]], axis=2)
    return emb_out, all_emb


# speedup vs baseline: 5.0554x; 5.0554x over previous
import functools

import jax
import jax.numpy as jnp
from jax import lax
from jax.experimental import pallas as pl
from jax.experimental.pallas import tpu as pltpu
from jax.experimental.pallas import tpu_sc as plsc

_N = 10000
_E = 160000
_DIN = 128
_DK = 64
_K = 2
_ITERS = 4
_INDEX = [[0, 1], [1, 0], [1, 2], [1, 3], [1, 4], [1, 5], [1, 6], [1, 7]]

_NW = 32
_EPAD = 163840
_EPW = _EPAD // _NW
_CH = 512
_NSUB = _CH // 128
_NCHUNK = _EPW // _CH
_NPAD = 10240
_RPT = _NPAD // 16
_TW = 80
_NTW = 16
_DUMMY = 10100


def _leaky(x):
    return jnp.where(x >= 0, x, 0.2 * x)


def _l2n_last(x):
    n = jnp.sqrt(jnp.sum(x * x, axis=-1, keepdims=True))
    return x / jnp.maximum(n, 1e-12)


def _fac_body(emb_ref, w_ref, out_ref):
    x = emb_ref[0]
    for k in range(_K):
        y = jnp.dot(x, w_ref[0, k], preferred_element_type=jnp.float32)
        out_ref[0, k] = _l2n_last(_leaky(y))


def _fac(emb, Wtk):
    bn = 1000
    return pl.pallas_call(
        _fac_body,
        grid=(8, _N // bn),
        in_specs=[
            pl.BlockSpec((1, bn, _DIN), lambda t, nb: (t, nb, 0)),
            pl.BlockSpec((1, _K, _DIN, _DK), lambda t, nb: (t, 0, 0, 0)),
        ],
        out_specs=pl.BlockSpec((1, _K, bn, _DK), lambda t, nb: (t, 0, nb, 0)),
        out_shape=jax.ShapeDtypeStruct((8, _K, _N, _DK), jnp.float32),
    )(emb, Wtk)


def _edge_pass(upad, ug, ig, ntab, otab0, otab1, zeros_tile):
    mesh = plsc.VectorSubcoreMesh(core_axis_name="c", subcore_axis_name="s")

    idx_scratch = [pltpu.VMEM((128,), jnp.int32) for _ in range(3 * _NSUB)]

    @functools.partial(
        pl.kernel,
        out_type=jax.ShapeDtypeStruct((8 * 2 * 2 * _NPAD, _TW), jnp.float32),
        mesh=mesh,
        compiler_params=pltpu.CompilerParams(use_tc_tiling_on_sc=False),
        scratch_types=idx_scratch + [
            pltpu.VMEM((_CH, _NTW), jnp.float32),
            pltpu.VMEM((_CH, _TW), jnp.float32),
            pltpu.VMEM((128, _TW), jnp.float32),
            pltpu.VMEM_SHARED((_NPAD, _TW), jnp.float32),
            pltpu.SemaphoreType.DMA,
            pltpu.SemaphoreType.DMA,
        ],
    )
    def k(upad_h, ug_h, ig_h, ntab_h, otab0_h, otab1_h, ztile_h, out_h, *refs):
        ubs = refs[0:_NSUB]
        ugs = refs[_NSUB:2 * _NSUB]
        igs = refs[2 * _NSUB:3 * _NSUB]
        nb, ob, zbuf, acc, sem1, sem2 = refs[3 * _NSUB:]
        cid = lax.axis_index("c")
        sid = lax.axis_index("s")
        wid = sid * 2 + cid
        pltpu.sync_copy(ztile_h, zbuf)

        for e in range(8):
            ebase = e * _EPAD + wid * _EPW
            for kk in range(2):
                otab_h = (otab0_h, otab1_h)[kk]
                for b in range(_RPT // 128):
                    pltpu.sync_copy(zbuf,
                                    acc.at[pl.ds(sid * _RPT + b * 128, 128)])
                plsc.subcore_barrier()

                def chunk_body(ci, _):
                    off = ebase + ci * _CH
                    for b in range(_NSUB):
                        sl = pl.ds(off + b * 128, 128)
                        pltpu.sync_copy(upad_h.at[sl], ubs[b])
                        pltpu.sync_copy(ug_h.at[sl], ugs[b])
                        pltpu.sync_copy(ig_h.at[sl], igs[b])
                    cps = []
                    for b in range(_NSUB):
                        cps.append(pltpu.async_copy(
                            ntab_h.at[ugs[b]],
                            nb.at[pl.ds(b * 128, 128)], sem1))
                        cps.append(pltpu.async_copy(
                            otab_h.at[igs[b]],
                            ob.at[pl.ds(b * 128, 128)], sem2))
                    for cp in cps:
                        cp.wait()

                    def scale(c, _):
                        nrow = nb[c, pl.ds(0, 16)]
                        orow = ob[c, pl.ds(_DK, 16)]
                        srel = jnp.maximum(nrow + orow, 0.0)
                        ev = srel[0] * nrow[2] + srel[1] * nrow[3]
                        wv = jnp.exp(jnp.full((16,), ev, jnp.float32))
                        for v in range(_TW // 16):
                            sl = pl.ds(v * 16, 16)
                            ob[c, sl] = ob[c, sl] * wv
                        return 0

                    lax.fori_loop(0, _CH, scale, 0)

                    for b in range(_NSUB):
                        pltpu.sync_copy(ob.at[pl.ds(b * 128, 128)],
                                        acc.at[ubs[b]], add=True)
                    return 0

                lax.fori_loop(0, _NCHUNK, chunk_body, 0)
                plsc.subcore_barrier()

                out_base = ((e * 2 + kk) * 2 + cid) * _NPAD + sid * _RPT
                for b in range(_RPT // 320):
                    pltpu.sync_copy(acc.at[pl.ds(sid * _RPT + b * 320, 320)],
                                    ob.at[pl.ds(0, 320)])
                    pltpu.sync_copy(ob.at[pl.ds(0, 320)],
                                    out_h.at[pl.ds(out_base + b * 320, 320)])
                plsc.subcore_barrier()

    out = k(upad, ug, ig, ntab, otab0, otab1, zeros_tile)
    return out.reshape(8, 2, 2, _NPAD, _TW)


def kernel(edge_list, emb, Wtk, at, W, q_rela):
    all_emb = _fac(emb, Wtk)
    a1 = at[:, :, :_DK]
    a2 = at[:, :, _DK:]
    r_rela = jnp.full((8, _K, _N), 1.0 / _K, dtype=jnp.float32)

    off8 = (jnp.arange(8, dtype=jnp.int32) * _N)[:, None]
    u = edge_list[:, 0, :]
    i = edge_list[:, 1, :]
    pad_u = jnp.full((8, _EPAD - _E), _DUMMY, jnp.int32)
    pad_g = jnp.zeros((8, _EPAD - _E), jnp.int32)
    upad = jnp.concatenate([u, pad_u], axis=1).reshape(-1)
    ug = jnp.concatenate([u + off8, pad_g], axis=1).reshape(-1)
    ig = jnp.concatenate([i + off8, pad_g], axis=1).reshape(-1)
    zeros_tile = jnp.zeros((128, _TW), jnp.float32)

    srcs = jnp.array([s for s, _ in _INDEX])
    dsts = jnp.array([d for _, d in _INDEX])
    ones_col = jnp.ones((8, _N, 1), jnp.float32)
    zpad_o = jnp.zeros((8, _N, _TW - _DK - 3), jnp.float32)
    zpad_n = jnp.zeros((8, _N, _NTW - 4), jnp.float32)

    for _ in range(_ITERS):
        ne = all_emb[srcs]
        oe = all_emb[dsts]
        pn = jnp.einsum('eknd,ekd->ekn', ne, a1)
        po = jnp.einsum('eknd,ekd->ekn', oe, a2)
        ntab = jnp.concatenate(
            [pn[:, 0, :, None], pn[:, 1, :, None],
             r_rela[:, 0, :, None], r_rela[:, 1, :, None], zpad_n],
            axis=2).reshape(8 * _N, _NTW)
        otab0 = jnp.concatenate(
            [oe[:, 0], po[:, 0, :, None], po[:, 1, :, None], ones_col,
             zpad_o], axis=2).reshape(8 * _N, _TW)
        otab1 = jnp.concatenate(
            [oe[:, 1], po[:, 0, :, None], po[:, 1, :, None], ones_col,
             zpad_o], axis=2).reshape(8 * _N, _TW)

        accs = _edge_pass(upad, ug, ig, ntab, otab0, otab1, zeros_tile)
        acc = (accs[:, :, 0, :_N] + accs[:, :, 1, :_N])
        s = jnp.maximum(acc[:, 0, :, _DK + 2], 1e-30)
        num = acc[:, :, :, :_DK]
        z = _leaky(num / s[:, None, :, None])
        emb_z = jnp.einsum('eknd,df->eknf', z, W)
        nr = jnp.einsum('eknd,ed->ekn', jnp.tanh(emb_z), q_rela)
        r_rela = jax.nn.softmax(nr, axis=1)

        ego0 = all_emb[0] + emb_z[0] * r_rela[0][:, :, None]
        ego1 = all_emb[1]
        for j in range(1, 8):
            ego1 = ego1 + emb_z[j] * r_rela[j][:, :, None]
        all_emb = all_emb.at[0].set(_l2n_last(ego0))
        all_emb = all_emb.at[1].set(_l2n_last(ego1))

    emb_out = jnp.concatenate([all_emb[:, 0], all_emb[:, 1]], axis=2)
    return emb_out, all_emb


# speedup vs baseline: 5.7281x; 1.1331x over previous
import functools

import jax
import jax.numpy as jnp
from jax import lax
from jax.experimental import pallas as pl
from jax.experimental.pallas import tpu as pltpu
from jax.experimental.pallas import tpu_sc as plsc

_N = 10000
_E = 160000
_DIN = 128
_DK = 64
_K = 2
_ITERS = 4
_INDEX = [[0, 1], [1, 0], [1, 2], [1, 3], [1, 4], [1, 5], [1, 6], [1, 7]]

_NW = 32
_EPAD = 163840
_EPW = _EPAD // _NW
_CH = 512
_NSUB = _CH // 128
_NCHUNK = _EPW // _CH
_NPAD = 10240
_RPT = _NPAD // 16
_TW = 80
_NTW = 16
_DUMMY = 10100


def _leaky(x):
    return jnp.where(x >= 0, x, 0.2 * x)


def _l2n_last(x):
    n = jnp.sqrt(jnp.sum(x * x, axis=-1, keepdims=True))
    return x / jnp.maximum(n, 1e-12)


def _fac_body(emb_ref, w_ref, out_ref):
    x = emb_ref[0]
    for k in range(_K):
        y = jnp.dot(x, w_ref[0, k], preferred_element_type=jnp.float32)
        out_ref[0, k] = _l2n_last(_leaky(y))


def _fac(emb, Wtk):
    bn = 1000
    return pl.pallas_call(
        _fac_body,
        grid=(8, _N // bn),
        in_specs=[
            pl.BlockSpec((1, bn, _DIN), lambda t, nb: (t, nb, 0)),
            pl.BlockSpec((1, _K, _DIN, _DK), lambda t, nb: (t, 0, 0, 0)),
        ],
        out_specs=pl.BlockSpec((1, _K, bn, _DK), lambda t, nb: (t, 0, nb, 0)),
        out_shape=jax.ShapeDtypeStruct((8, _K, _N, _DK), jnp.float32),
    )(emb, Wtk)


def _edge_pass(upad, ug, ig, ntab, otab0, otab1, zeros_tile):
    mesh = plsc.VectorSubcoreMesh(core_axis_name="c", subcore_axis_name="s")
    idx_rows = _EPW // 128

    @functools.partial(
        pl.kernel,
        out_type=jax.ShapeDtypeStruct((8 * 2 * 2 * _NPAD, _TW), jnp.float32),
        mesh=mesh,
        compiler_params=pltpu.CompilerParams(use_tc_tiling_on_sc=False),
        scratch_types=[
            pltpu.VMEM((idx_rows, 128), jnp.int32),
            pltpu.VMEM((idx_rows, 128), jnp.int32),
            pltpu.VMEM((idx_rows, 128), jnp.int32),
            pltpu.VMEM((_CH, _NTW), jnp.float32),
            pltpu.VMEM((_CH, _TW), jnp.float32),
            pltpu.VMEM((128, _TW), jnp.float32),
            pltpu.VMEM_SHARED((_NPAD, _TW), jnp.float32),
            pltpu.SemaphoreType.DMA,
            pltpu.SemaphoreType.DMA,
        ],
    )
    def k(upad_h, ug_h, ig_h, ntab_h, otab0_h, otab1_h, ztile_h, out_h,
          ubuf, ugbuf, igbuf, nb, ob, zbuf, acc, sem1, sem2):
        cid = lax.axis_index("c")
        sid = lax.axis_index("s")
        wid = sid * 2 + cid
        pltpu.sync_copy(ztile_h, zbuf)

        for e in range(8):
            erow = (e * _EPAD + wid * _EPW) // 128
            rs = pl.ds(erow, idx_rows)
            i1 = pltpu.async_copy(upad_h.at[rs], ubuf, sem1)
            i2 = pltpu.async_copy(ug_h.at[rs], ugbuf, sem1)
            i3 = pltpu.async_copy(ig_h.at[rs], igbuf, sem1)
            i1.wait()
            i2.wait()
            i3.wait()
            for kk in range(2):
                otab_h = (otab0_h, otab1_h)[kk]
                for b in range(_RPT // 128):
                    pltpu.sync_copy(zbuf,
                                    acc.at[pl.ds(sid * _RPT + b * 128, 128)])
                plsc.subcore_barrier()

                def chunk_body(ci, _):
                    cps = []
                    for b in range(_NSUB):
                        r = ci * _NSUB + b
                        cps.append(pltpu.async_copy(
                            ntab_h.at[ugbuf.at[r]],
                            nb.at[pl.ds(b * 128, 128)], sem1))
                        cps.append(pltpu.async_copy(
                            otab_h.at[igbuf.at[r]],
                            ob.at[pl.ds(b * 128, 128)], sem2))
                    for cp in cps:
                        cp.wait()

                    def scale(c, _):
                        nrow = nb[c, pl.ds(0, 16)]
                        orow = ob[c, pl.ds(_DK, 16)]
                        srel = jnp.maximum(nrow + orow, 0.0)
                        ev = srel[0] * nrow[2] + srel[1] * nrow[3]
                        wv = jnp.exp(jnp.full((16,), ev, jnp.float32))
                        for v in range(_TW // 16):
                            sl = pl.ds(v * 16, 16)
                            ob[c, sl] = ob[c, sl] * wv
                        return 0

                    lax.fori_loop(0, _CH, scale, 0)

                    for b in range(_NSUB):
                        pltpu.sync_copy(ob.at[pl.ds(b * 128, 128)],
                                        acc.at[ubuf.at[ci * _NSUB + b]],
                                        add=True)
                    return 0

                lax.fori_loop(0, _NCHUNK, chunk_body, 0)
                plsc.subcore_barrier()

                out_base = ((e * 2 + kk) * 2 + cid) * _NPAD + sid * _RPT
                for b in range(_RPT // 320):
                    pltpu.sync_copy(acc.at[pl.ds(sid * _RPT + b * 320, 320)],
                                    ob.at[pl.ds(0, 320)])
                    pltpu.sync_copy(ob.at[pl.ds(0, 320)],
                                    out_h.at[pl.ds(out_base + b * 320, 320)])
                plsc.subcore_barrier()

    out = k(upad, ug, ig, ntab, otab0, otab1, zeros_tile)
    return out.reshape(8, 2, 2, _NPAD, _TW)


def kernel(edge_list, emb, Wtk, at, W, q_rela):
    all_emb = _fac(emb, Wtk)
    a1 = at[:, :, :_DK]
    a2 = at[:, :, _DK:]
    r_rela = jnp.full((8, _K, _N), 1.0 / _K, dtype=jnp.float32)

    off8 = (jnp.arange(8, dtype=jnp.int32) * _N)[:, None]
    u = edge_list[:, 0, :]
    i = edge_list[:, 1, :]
    pad_u = jnp.full((8, _EPAD - _E), _DUMMY, jnp.int32)
    pad_g = jnp.zeros((8, _EPAD - _E), jnp.int32)
    upad = jnp.concatenate([u, pad_u], axis=1).reshape(-1, 128)
    ug = jnp.concatenate([u + off8, pad_g], axis=1).reshape(-1, 128)
    ig = jnp.concatenate([i + off8, pad_g], axis=1).reshape(-1, 128)
    zeros_tile = jnp.zeros((128, _TW), jnp.float32)

    srcs = jnp.array([s for s, _ in _INDEX])
    dsts = jnp.array([d for _, d in _INDEX])
    ones_col = jnp.ones((8, _N, 1), jnp.float32)
    zpad_o = jnp.zeros((8, _N, _TW - _DK - 3), jnp.float32)
    zpad_n = jnp.zeros((8, _N, _NTW - 4), jnp.float32)

    for _ in range(_ITERS):
        ne = all_emb[srcs]
        oe = all_emb[dsts]
        pn = jnp.einsum('eknd,ekd->ekn', ne, a1)
        po = jnp.einsum('eknd,ekd->ekn', oe, a2)
        ntab = jnp.concatenate(
            [pn[:, 0, :, None], pn[:, 1, :, None],
             r_rela[:, 0, :, None], r_rela[:, 1, :, None], zpad_n],
            axis=2).reshape(8 * _N, _NTW)
        otab0 = jnp.concatenate(
            [oe[:, 0], po[:, 0, :, None], po[:, 1, :, None], ones_col,
             zpad_o], axis=2).reshape(8 * _N, _TW)
        otab1 = jnp.concatenate(
            [oe[:, 1], po[:, 0, :, None], po[:, 1, :, None], ones_col,
             zpad_o], axis=2).reshape(8 * _N, _TW)

        accs = _edge_pass(upad, ug, ig, ntab, otab0, otab1, zeros_tile)
        acc = (accs[:, :, 0, :_N] + accs[:, :, 1, :_N])
        s = jnp.maximum(acc[:, 0, :, _DK + 2], 1e-30)
        num = acc[:, :, :, :_DK]
        z = _leaky(num / s[:, None, :, None])
        emb_z = jnp.einsum('eknd,df->eknf', z, W)
        nr = jnp.einsum('eknd,ed->ekn', jnp.tanh(emb_z), q_rela)
        r_rela = jax.nn.softmax(nr, axis=1)

        ego0 = all_emb[0] + emb_z[0] * r_rela[0][:, :, None]
        ego1 = all_emb[1]
        for j in range(1, 8):
            ego1 = ego1 + emb_z[j] * r_rela[j][:, :, None]
        all_emb = all_emb.at[0].set(_l2n_last(ego0))
        all_emb = all_emb.at[1].set(_l2n_last(ego1))

    emb_out = jnp.concatenate([all_emb[:, 0], all_emb[:, 1]], axis=2)
    return emb_out, all_emb


# speedup vs baseline: 5.7868x; 1.0102x over previous
import functools

import jax
import jax.numpy as jnp
from jax import lax
from jax.experimental import pallas as pl
from jax.experimental.pallas import tpu as pltpu
from jax.experimental.pallas import tpu_sc as plsc

_N = 10000
_E = 160000
_DIN = 128
_DK = 64
_K = 2
_ITERS = 4
_INDEX = [[0, 1], [1, 0], [1, 2], [1, 3], [1, 4], [1, 5], [1, 6], [1, 7]]

_NW = 32
_EPAD = 163840
_EPW = _EPAD // _NW
_CH = 256
_NSUB = _CH // 128
_NCHUNK = _EPW // _CH
_NPAD = 10240
_RPT = _NPAD // 16
_TW = 80
_NTW = 16
_DUMMY = 10100


def _leaky(x):
    return jnp.where(x >= 0, x, 0.2 * x)


def _l2n_last(x):
    n = jnp.sqrt(jnp.sum(x * x, axis=-1, keepdims=True))
    return x / jnp.maximum(n, 1e-12)


def _fac_body(emb_ref, w_ref, out_ref):
    x = emb_ref[0]
    for k in range(_K):
        y = jnp.dot(x, w_ref[0, k], preferred_element_type=jnp.float32)
        out_ref[0, k] = _l2n_last(_leaky(y))


def _fac(emb, Wtk):
    bn = 1000
    return pl.pallas_call(
        _fac_body,
        grid=(8, _N // bn),
        in_specs=[
            pl.BlockSpec((1, bn, _DIN), lambda t, nb: (t, nb, 0)),
            pl.BlockSpec((1, _K, _DIN, _DK), lambda t, nb: (t, 0, 0, 0)),
        ],
        out_specs=pl.BlockSpec((1, _K, bn, _DK), lambda t, nb: (t, 0, nb, 0)),
        out_shape=jax.ShapeDtypeStruct((8, _K, _N, _DK), jnp.float32),
    )(emb, Wtk)


def _edge_pass(upad, ug, ig, ntab, otab0, otab1, zeros_tile):
    mesh = plsc.VectorSubcoreMesh(core_axis_name="c", subcore_axis_name="s")
    idx_rows = _EPW // 128

    @functools.partial(
        pl.kernel,
        out_type=jax.ShapeDtypeStruct((8 * 2 * 2 * _NPAD, _TW), jnp.float32),
        mesh=mesh,
        compiler_params=pltpu.CompilerParams(use_tc_tiling_on_sc=False),
        scratch_types=[
            pltpu.VMEM((idx_rows, 128), jnp.int32),
            pltpu.VMEM((idx_rows, 128), jnp.int32),
            pltpu.VMEM((idx_rows, 128), jnp.int32),
            pltpu.VMEM((_CH, _NTW), jnp.float32),
            pltpu.VMEM((_CH, _TW), jnp.float32),
            pltpu.VMEM((_CH, _NTW), jnp.float32),
            pltpu.VMEM((_CH, _TW), jnp.float32),
            pltpu.VMEM((128, _TW), jnp.float32),
            pltpu.VMEM_SHARED((_NPAD, _TW), jnp.float32),
            pltpu.SemaphoreType.DMA,
            pltpu.SemaphoreType.DMA,
            pltpu.SemaphoreType.DMA,
            pltpu.SemaphoreType.DMA,
        ],
    )
    def k(upad_h, ug_h, ig_h, ntab_h, otab0_h, otab1_h, ztile_h, out_h,
          ubuf, ugbuf, igbuf, nbA, obA, nbB, obB, zbuf, acc,
          semA, semB, semC, semD):
        cid = lax.axis_index("c")
        sid = lax.axis_index("s")
        wid = sid * 2 + cid
        pltpu.sync_copy(ztile_h, zbuf)

        for e in range(8):
            erow = (e * _EPAD + wid * _EPW) // 128
            rs = pl.ds(erow, idx_rows)
            i1 = pltpu.async_copy(upad_h.at[rs], ubuf, semA)
            i2 = pltpu.async_copy(ug_h.at[rs], ugbuf, semA)
            i3 = pltpu.async_copy(ig_h.at[rs], igbuf, semA)
            i1.wait()
            i2.wait()
            i3.wait()
            for kk in range(2):
                otab_h = (otab0_h, otab1_h)[kk]
                for b in range(_RPT // 128):
                    pltpu.sync_copy(zbuf,
                                    acc.at[pl.ds(sid * _RPT + b * 128, 128)])
                plsc.subcore_barrier()

                def issue(ci, nb, ob, semN, semO):
                    cps = []
                    for b in range(_NSUB):
                        r = ci * _NSUB + b
                        cps.append(pltpu.async_copy(
                            ntab_h.at[ugbuf.at[r]],
                            nb.at[pl.ds(b * 128, 128)], semN))
                        cps.append(pltpu.async_copy(
                            otab_h.at[igbuf.at[r]],
                            ob.at[pl.ds(b * 128, 128)], semO))
                    return cps

                def process(ci, nb, ob):
                    def scale(c, _):
                        nrow = nb[c, pl.ds(0, 16)]
                        orow = ob[c, pl.ds(_DK, 16)]
                        srel = jnp.maximum(nrow + orow, 0.0)
                        ev = srel[0] * nrow[2] + srel[1] * nrow[3]
                        wv = jnp.exp(jnp.full((16,), ev, jnp.float32))
                        for v in range(_TW // 16):
                            sl = pl.ds(v * 16, 16)
                            ob[c, sl] = ob[c, sl] * wv
                        return 0

                    lax.fori_loop(0, _CH, scale, 0)

                    for b in range(_NSUB):
                        pltpu.sync_copy(ob.at[pl.ds(b * 128, 128)],
                                        acc.at[ubuf.at[ci * _NSUB + b]],
                                        add=True)

                def pair_body(t, _):
                    cA = 2 * t
                    cB = 2 * t + 1
                    gA = issue(cA, nbA, obA, semA, semB)
                    gB = issue(cB, nbB, obB, semC, semD)
                    for cp in gA:
                        cp.wait()
                    process(cA, nbA, obA)
                    for cp in gB:
                        cp.wait()
                    process(cB, nbB, obB)
                    return 0

                lax.fori_loop(0, _NCHUNK // 2, pair_body, 0)
                plsc.subcore_barrier()

                out_base = ((e * 2 + kk) * 2 + cid) * _NPAD + sid * _RPT
                for b in range(_RPT // 320):
                    pltpu.sync_copy(acc.at[pl.ds(sid * _RPT + b * 320, 320)],
                                    obA.at[pl.ds(0, 320)])
                    pltpu.sync_copy(obA.at[pl.ds(0, 320)],
                                    out_h.at[pl.ds(out_base + b * 320, 320)])
                plsc.subcore_barrier()

    out = k(upad, ug, ig, ntab, otab0, otab1, zeros_tile)
    return out.reshape(8, 2, 2, _NPAD, _TW)


def kernel(edge_list, emb, Wtk, at, W, q_rela):
    all_emb = _fac(emb, Wtk)
    a1 = at[:, :, :_DK]
    a2 = at[:, :, _DK:]
    r_rela = jnp.full((8, _K, _N), 1.0 / _K, dtype=jnp.float32)

    off8 = (jnp.arange(8, dtype=jnp.int32) * _N)[:, None]
    u = edge_list[:, 0, :]
    i = edge_list[:, 1, :]
    pad_u = jnp.full((8, _EPAD - _E), _DUMMY, jnp.int32)
    pad_g = jnp.zeros((8, _EPAD - _E), jnp.int32)
    upad = jnp.concatenate([u, pad_u], axis=1).reshape(-1, 128)
    ug = jnp.concatenate([u + off8, pad_g], axis=1).reshape(-1, 128)
    ig = jnp.concatenate([i + off8, pad_g], axis=1).reshape(-1, 128)
    zeros_tile = jnp.zeros((128, _TW), jnp.float32)

    srcs = jnp.array([s for s, _ in _INDEX])
    dsts = jnp.array([d for _, d in _INDEX])
    ones_col = jnp.ones((8, _N, 1), jnp.float32)
    zpad_o = jnp.zeros((8, _N, _TW - _DK - 3), jnp.float32)
    zpad_n = jnp.zeros((8, _N, _NTW - 4), jnp.float32)

    for _ in range(_ITERS):
        ne = all_emb[srcs]
        oe = all_emb[dsts]
        pn = jnp.einsum('eknd,ekd->ekn', ne, a1)
        po = jnp.einsum('eknd,ekd->ekn', oe, a2)
        ntab = jnp.concatenate(
            [pn[:, 0, :, None], pn[:, 1, :, None],
             r_rela[:, 0, :, None], r_rela[:, 1, :, None], zpad_n],
            axis=2).reshape(8 * _N, _NTW)
        otab0 = jnp.concatenate(
            [oe[:, 0], po[:, 0, :, None], po[:, 1, :, None], ones_col,
             zpad_o], axis=2).reshape(8 * _N, _TW)
        otab1 = jnp.concatenate(
            [oe[:, 1], po[:, 0, :, None], po[:, 1, :, None], ones_col,
             zpad_o], axis=2).reshape(8 * _N, _TW)

        accs = _edge_pass(upad, ug, ig, ntab, otab0, otab1, zeros_tile)
        acc = (accs[:, :, 0, :_N] + accs[:, :, 1, :_N])
        s = jnp.maximum(acc[:, 0, :, _DK + 2], 1e-30)
        num = acc[:, :, :, :_DK]
        z = _leaky(num / s[:, None, :, None])
        emb_z = jnp.einsum('eknd,df->eknf', z, W)
        nr = jnp.einsum('eknd,ed->ekn', jnp.tanh(emb_z), q_rela)
        r_rela = jax.nn.softmax(nr, axis=1)

        ego0 = all_emb[0] + emb_z[0] * r_rela[0][:, :, None]
        ego1 = all_emb[1]
        for j in range(1, 8):
            ego1 = ego1 + emb_z[j] * r_rela[j][:, :, None]
        all_emb = all_emb.at[0].set(_l2n_last(ego0))
        all_emb = all_emb.at[1].set(_l2n_last(ego1))

    emb_out = jnp.concatenate([all_emb[:, 0], all_emb[:, 1]], axis=2)
    return emb_out, all_emb


# speedup vs baseline: 6.5262x; 1.1278x over previous
import functools

import jax
import jax.numpy as jnp
from jax import lax
from jax.experimental import pallas as pl
from jax.experimental.pallas import tpu as pltpu
from jax.experimental.pallas import tpu_sc as plsc

_N = 10000
_E = 160000
_DIN = 128
_DK = 64
_K = 2
_ITERS = 4
_INDEX = [[0, 1], [1, 0], [1, 2], [1, 3], [1, 4], [1, 5], [1, 6], [1, 7]]

_NW = 32
_EPAD = 163840
_EPW = _EPAD // _NW
_CH = 256
_NSUB = _CH // 128
_NCHUNK = _EPW // _CH
_NPAD = 10240
_RPT = _NPAD // 16
_TW = 80
_NTW = 16
_DUMMY = 10100


def _leaky(x):
    return jnp.where(x >= 0, x, 0.2 * x)


def _l2n_last(x):
    n = jnp.sqrt(jnp.sum(x * x, axis=-1, keepdims=True))
    return x / jnp.maximum(n, 1e-12)


def _fac_body(emb_ref, w_ref, out_ref):
    x = emb_ref[0]
    for k in range(_K):
        y = jnp.dot(x, w_ref[0, k], preferred_element_type=jnp.float32)
        out_ref[0, k] = _l2n_last(_leaky(y))


def _fac(emb, Wtk):
    bn = 1000
    return pl.pallas_call(
        _fac_body,
        grid=(8, _N // bn),
        in_specs=[
            pl.BlockSpec((1, bn, _DIN), lambda t, nb: (t, nb, 0)),
            pl.BlockSpec((1, _K, _DIN, _DK), lambda t, nb: (t, 0, 0, 0)),
        ],
        out_specs=pl.BlockSpec((1, _K, bn, _DK), lambda t, nb: (t, 0, nb, 0)),
        out_shape=jax.ShapeDtypeStruct((8, _K, _N, _DK), jnp.float32),
    )(emb, Wtk)


def _tables_body(ne_ref, oe_ref, a1_ref, a2_ref, r_ref, nt_ref, o0_ref,
                 o1_ref):
    e = pl.program_id(0)
    cols = []
    for k in range(_K):
        pn = jnp.dot(ne_ref[0, k], a1_ref[0, k],
                     preferred_element_type=jnp.float32)
        cols.append(pn[:, None])
    rt = r_ref[0]
    for k in range(_K):
        cols.append(rt[:, k:k + 1])
    bn = cols[0].shape[0]
    nt_ref[0] = jnp.concatenate(
        cols + [jnp.zeros((bn, _NTW - 4), jnp.float32)], axis=1)
    po = [jnp.dot(oe_ref[0, k], a2_ref[0, k],
                  preferred_element_type=jnp.float32)[:, None]
          for k in range(_K)]
    tail = po + [jnp.ones((bn, 1), jnp.float32),
                 jnp.zeros((bn, _TW - _DK - 3), jnp.float32)]
    o0_ref[0] = jnp.concatenate([oe_ref[0, 0]] + tail, axis=1)
    o1_ref[0] = jnp.concatenate([oe_ref[0, 1]] + tail, axis=1)
    del e


def _tables(ne, oe, a1, a2, r_rela):
    bn = 1000
    grid = (8, _N // bn)
    nt, o0, o1 = pl.pallas_call(
        _tables_body,
        grid=grid,
        in_specs=[
            pl.BlockSpec((1, _K, bn, _DK), lambda e, nb: (e, 0, nb, 0)),
            pl.BlockSpec((1, _K, bn, _DK), lambda e, nb: (e, 0, nb, 0)),
            pl.BlockSpec((1, _K, _DK), lambda e, nb: (e, 0, 0)),
            pl.BlockSpec((1, _K, _DK), lambda e, nb: (e, 0, 0)),
            pl.BlockSpec((1, bn, _K), lambda e, nb: (e, nb, 0)),
        ],
        out_specs=[
            pl.BlockSpec((1, bn, _NTW), lambda e, nb: (e, nb, 0)),
            pl.BlockSpec((1, bn, _TW), lambda e, nb: (e, nb, 0)),
            pl.BlockSpec((1, bn, _TW), lambda e, nb: (e, nb, 0)),
        ],
        out_shape=[
            jax.ShapeDtypeStruct((8, _N, _NTW), jnp.float32),
            jax.ShapeDtypeStruct((8, _N, _TW), jnp.float32),
            jax.ShapeDtypeStruct((8, _N, _TW), jnp.float32),
        ],
    )(ne, oe, a1, a2, r_rela)
    return (nt.reshape(8 * _N, _NTW), o0.reshape(8 * _N, _TW),
            o1.reshape(8 * _N, _TW))


def _post_body(acc_ref, emb01_ref, w_ref, q_ref, out01_ref, r_ref):
    accm = acc_ref[:, :, 0] + acc_ref[:, :, 1]
    s = jnp.maximum(accm[:, 0, :, _DK + 2], 1e-30)
    num = accm[:, :, :, :_DK]
    z = _leaky(num / s[:, None, :, None])
    bn = z.shape[2]
    emb_z = jnp.dot(z.reshape(8 * _K * bn, _DK), w_ref[...],
                    preferred_element_type=jnp.float32)
    emb_z = emb_z.reshape(8, _K, bn, _DK)
    q = q_ref[...]
    nr = jnp.sum(jnp.tanh(emb_z) * q[:, None, None, :], axis=3)
    m = jnp.max(nr, axis=1, keepdims=True)
    ex = jnp.exp(nr - m)
    r = ex / jnp.sum(ex, axis=1, keepdims=True)
    ego0 = emb01_ref[0] + emb_z[0] * r[0][:, :, None]
    ego1 = emb01_ref[1]
    for j in range(1, 8):
        ego1 = ego1 + emb_z[j] * r[j][:, :, None]
    out01_ref[0] = _l2n_last(ego0)
    out01_ref[1] = _l2n_last(ego1)
    r_ref[...] = r.transpose(0, 2, 1)


def _post(accs, emb01, W, q_rela):
    bn = 400
    grid = (_N // bn,)
    return pl.pallas_call(
        _post_body,
        grid=grid,
        in_specs=[
            pl.BlockSpec((8, _K, 2, bn, _TW), lambda nb: (0, 0, 0, nb, 0)),
            pl.BlockSpec((2, _K, bn, _DK), lambda nb: (0, 0, nb, 0)),
            pl.BlockSpec((_DK, _DK), lambda nb: (0, 0)),
            pl.BlockSpec((8, _DK), lambda nb: (0, 0)),
        ],
        out_specs=[
            pl.BlockSpec((2, _K, bn, _DK), lambda nb: (0, 0, nb, 0)),
            pl.BlockSpec((8, bn, _K), lambda nb: (0, nb, 0)),
        ],
        out_shape=[
            jax.ShapeDtypeStruct((2, _K, _N, _DK), jnp.float32),
            jax.ShapeDtypeStruct((8, _N, _K), jnp.float32),
        ],
    )(accs, emb01, W, q_rela)


def _edge_pass(upad, ug, ig, ntab, otab0, otab1, zeros_tile):
    mesh = plsc.VectorSubcoreMesh(core_axis_name="c", subcore_axis_name="s")
    idx_rows = _EPW // 128

    @functools.partial(
        pl.kernel,
        out_type=jax.ShapeDtypeStruct((8 * 2 * 2 * _NPAD, _TW), jnp.float32),
        mesh=mesh,
        compiler_params=pltpu.CompilerParams(use_tc_tiling_on_sc=False),
        scratch_types=[
            pltpu.VMEM((idx_rows, 128), jnp.int32),
            pltpu.VMEM((idx_rows, 128), jnp.int32),
            pltpu.VMEM((idx_rows, 128), jnp.int32),
            pltpu.VMEM((_CH, _NTW), jnp.float32),
            pltpu.VMEM((_CH, _TW), jnp.float32),
            pltpu.VMEM((_CH, _NTW), jnp.float32),
            pltpu.VMEM((_CH, _TW), jnp.float32),
            pltpu.VMEM((128, _TW), jnp.float32),
            pltpu.VMEM_SHARED((_NPAD, _TW), jnp.float32),
            pltpu.SemaphoreType.DMA,
            pltpu.SemaphoreType.DMA,
            pltpu.SemaphoreType.DMA,
            pltpu.SemaphoreType.DMA,
        ],
    )
    def k(upad_h, ug_h, ig_h, ntab_h, otab0_h, otab1_h, ztile_h, out_h,
          ubuf, ugbuf, igbuf, nbA, obA, nbB, obB, zbuf, acc,
          semA, semB, semC, semD):
        cid = lax.axis_index("c")
        sid = lax.axis_index("s")
        wid = sid * 2 + cid
        pltpu.sync_copy(ztile_h, zbuf)

        for e in range(8):
            erow = (e * _EPAD + wid * _EPW) // 128
            rs = pl.ds(erow, idx_rows)
            i1 = pltpu.async_copy(upad_h.at[rs], ubuf, semA)
            i2 = pltpu.async_copy(ug_h.at[rs], ugbuf, semA)
            i3 = pltpu.async_copy(ig_h.at[rs], igbuf, semA)
            i1.wait()
            i2.wait()
            i3.wait()
            for kk in range(2):
                otab_h = (otab0_h, otab1_h)[kk]
                for b in range(_RPT // 128):
                    pltpu.sync_copy(zbuf,
                                    acc.at[pl.ds(sid * _RPT + b * 128, 128)])
                plsc.subcore_barrier()

                def issue(ci, nb, ob, semN, semO):
                    cps = []
                    for b in range(_NSUB):
                        r = ci * _NSUB + b
                        cps.append(pltpu.async_copy(
                            ntab_h.at[ugbuf.at[r]],
                            nb.at[pl.ds(b * 128, 128)], semN))
                        cps.append(pltpu.async_copy(
                            otab_h.at[igbuf.at[r]],
                            ob.at[pl.ds(b * 128, 128)], semO))
                    return cps

                def process(ci, nb, ob):
                    def scale(c, _):
                        nrow = nb[c, pl.ds(0, 16)]
                        orow = ob[c, pl.ds(_DK, 16)]
                        srel = jnp.maximum(nrow + orow, 0.0)
                        ev = srel[0] * nrow[2] + srel[1] * nrow[3]
                        wv = jnp.exp(jnp.full((16,), ev, jnp.float32))
                        for v in range(_TW // 16):
                            sl = pl.ds(v * 16, 16)
                            ob[c, sl] = ob[c, sl] * wv
                        return 0

                    lax.fori_loop(0, _CH, scale, 0)

                    for b in range(_NSUB):
                        pltpu.sync_copy(ob.at[pl.ds(b * 128, 128)],
                                        acc.at[ubuf.at[ci * _NSUB + b]],
                                        add=True)

                def pair_body(t, _):
                    cA = 2 * t
                    cB = 2 * t + 1
                    gA = issue(cA, nbA, obA, semA, semB)
                    gB = issue(cB, nbB, obB, semC, semD)
                    for cp in gA:
                        cp.wait()
                    process(cA, nbA, obA)
                    for cp in gB:
                        cp.wait()
                    process(cB, nbB, obB)
                    return 0

                lax.fori_loop(0, _NCHUNK // 2, pair_body, 0)
                plsc.subcore_barrier()

                out_base = ((e * 2 + kk) * 2 + cid) * _NPAD + sid * _RPT
                for b in range(_RPT // 320):
                    pltpu.sync_copy(acc.at[pl.ds(sid * _RPT + b * 320, 320)],
                                    obA.at[pl.ds(0, 320)])
                    pltpu.sync_copy(obA.at[pl.ds(0, 320)],
                                    out_h.at[pl.ds(out_base + b * 320, 320)])
                plsc.subcore_barrier()

    out = k(upad, ug, ig, ntab, otab0, otab1, zeros_tile)
    return out.reshape(8, 2, 2, _NPAD, _TW)


def kernel(edge_list, emb, Wtk, at, W, q_rela):
    all_emb = _fac(emb, Wtk)
    a1 = at[:, :, :_DK]
    a2 = at[:, :, _DK:]
    r_rela = jnp.full((8, _N, _K), 1.0 / _K, dtype=jnp.float32)

    off8 = (jnp.arange(8, dtype=jnp.int32) * _N)[:, None]
    u = edge_list[:, 0, :]
    i = edge_list[:, 1, :]
    pad_u = jnp.full((8, _EPAD - _E), _DUMMY, jnp.int32)
    pad_g = jnp.zeros((8, _EPAD - _E), jnp.int32)
    upad = jnp.concatenate([u, pad_u], axis=1).reshape(-1, 128)
    ug = jnp.concatenate([u + off8, pad_g], axis=1).reshape(-1, 128)
    ig = jnp.concatenate([i + off8, pad_g], axis=1).reshape(-1, 128)
    zeros_tile = jnp.zeros((128, _TW), jnp.float32)

    srcs = jnp.array([s for s, _ in _INDEX])
    dsts = jnp.array([d for _, d in _INDEX])

    for _ in range(_ITERS):
        ne = all_emb[srcs]
        oe = all_emb[dsts]
        ntab, otab0, otab1 = _tables(ne, oe, a1, a2, r_rela)
        accs = _edge_pass(upad, ug, ig, ntab, otab0, otab1, zeros_tile)
        new01, r_rela = _post(accs[:, :, :, :_N], all_emb[:2], W, q_rela)
        all_emb = all_emb.at[0:2].set(new01)

    emb_out = jnp.concatenate([all_emb[:, 0], all_emb[:, 1]], axis=2)
    return emb_out, all_emb
